# Initial kernel scaffold; baseline (speedup 1.0000x reference)
import functools

import jax
import jax.numpy as jnp
from jax import lax
from jax.experimental import pallas as pl
from jax.experimental.pallas import tpu as pltpu
from jax.experimental.pallas import tpu_sc as plsc

N = 100000
E = 1600000
H = 32
T = 10

FB = 16
NBLK = 2 * T
CH = 80
GRP = 20
NCHUNK = E // CH
ROWS_PER_TILE = N // 16


def _sc_mesh():
    return plsc.VectorSubcoreMesh(
        core_axis_name="c", subcore_axis_name="s", num_cores=2, num_subcores=16
    )


def _aggregate_pass(table_hbm, src_r, dst_r, acc, zeros_hbm, out_slice,
                    src_buf, dst_buf, rows_buf, sem, sid, chunk0, ngroups):
    row0 = sid * ROWS_PER_TILE
    pltpu.sync_copy(zeros_hbm, acc.at[pl.ds(row0, ROWS_PER_TILE)])
    plsc.subcore_barrier()

    def group_body(g, _):
        c0 = chunk0 + g * GRP
        pltpu.sync_copy(src_r.at[pl.ds(c0, GRP)], src_buf)
        pltpu.sync_copy(dst_r.at[pl.ds(c0, GRP)], dst_buf)

        def fire(j, _):
            pltpu.async_copy(table_hbm.at[src_buf.at[j]], rows_buf.at[j], sem)
            return 0

        lax.fori_loop(0, GRP, fire, 0)

        def drain(j, _):
            pltpu.make_async_copy(
                table_hbm.at[src_buf.at[j]], rows_buf.at[j], sem
            ).wait()
            pltpu.sync_copy(rows_buf.at[j], acc.at[dst_buf.at[j]], add=True)
            return 0

        lax.fori_loop(0, GRP, drain, 0)
        return 0

    lax.fori_loop(0, ngroups, group_body, 0)
    plsc.subcore_barrier()
    pltpu.sync_copy(acc.at[pl.ds(row0, ROWS_PER_TILE)],
                    out_slice.at[pl.ds(row0, ROWS_PER_TILE)])
    plsc.subcore_barrier()


def _sc_phase1(xpad, src_r, dst_r, zeros_init):
    @functools.partial(
        pl.kernel,
        out_type=jax.ShapeDtypeStruct((2, N, FB), jnp.float32),
        mesh=_sc_mesh(),
        scratch_types=[
            pltpu.VMEM_SHARED((N, FB), jnp.float32),
            pltpu.VMEM((GRP, CH), jnp.int32),
            pltpu.VMEM((GRP, 1, CH), jnp.int32),
            pltpu.VMEM((GRP, CH, FB), jnp.float32),
            pltpu.SemaphoreType.DMA,
        ],
    )
    def k(table_hbm, src_hbm, dst_hbm, zeros_hbm, out_hbm,
          acc, src_buf, dst_buf, rows_buf, sem):
        cid = lax.axis_index("c")
        sid = lax.axis_index("s")
        chunks_per_tile = NCHUNK // 2 // 16
        chunk0 = cid * (NCHUNK // 2) + sid * chunks_per_tile
        _aggregate_pass(table_hbm, src_hbm, dst_hbm, acc, zeros_hbm,
                        out_hbm.at[cid], src_buf, dst_buf, rows_buf, sem,
                        sid, chunk0, chunks_per_tile // GRP)

    return k(xpad, src_r, dst_r, zeros_init)


def _sc_phase3(h1b, src_r, dst_r, zeros_init):
    @functools.partial(
        pl.kernel,
        out_type=jax.ShapeDtypeStruct((NBLK, N, FB), jnp.float32),
        mesh=_sc_mesh(),
        scratch_types=[
            pltpu.VMEM_SHARED((N, FB), jnp.float32),
            pltpu.VMEM((GRP, CH), jnp.int32),
            pltpu.VMEM((GRP, 1, CH), jnp.int32),
            pltpu.VMEM((GRP, CH, FB), jnp.float32),
            pltpu.SemaphoreType.DMA,
        ],
    )
    def k(table_hbm, src_hbm, dst_hbm, zeros_hbm, out_hbm,
          acc, src_buf, dst_buf, rows_buf, sem):
        cid = lax.axis_index("c")
        sid = lax.axis_index("s")
        chunks_per_tile = NCHUNK // 16
        chunk0 = sid * chunks_per_tile
        for j in range(T):
            b = 2 * j
            _aggregate_pass(table_hbm.at[b + cid], src_hbm, dst_hbm, acc,
                            zeros_hbm, out_hbm.at[b + cid], src_buf, dst_buf,
                            rows_buf, sem, sid, chunk0,
                            chunks_per_tile // GRP)

    return k(h1b, src_r, dst_r, zeros_init)


def _tc_phase2(xpad, m1p, W1l, W1r, b1):
    R = 2000

    def body(xp_ref, p_ref, wl_ref, wr_ref, b_ref, out_ref):
        p = p_ref[0] + p_ref[1]
        inv = 1.0 / jnp.maximum(p[:, 15:16], 1.0)
        wl = wl_ref[0][None, :]
        wr = wr_ref[0][None, :]
        bb = b_ref[0][None, :]
        for t in range(T):
            m = p[:, t:t + 1] * inv
            xt = xp_ref[:, t:t + 1]
            h = jnp.maximum(m * wl + xt * wr + bb, 0.0)
            out_ref[2 * t] = h[:, :FB]
            out_ref[2 * t + 1] = h[:, FB:]

    return pl.pallas_call(
        body,
        grid=(N // R,),
        in_specs=[
            pl.BlockSpec((R, FB), lambda i: (i, 0)),
            pl.BlockSpec((2, R, FB), lambda i: (0, i, 0)),
            pl.BlockSpec((1, H), lambda i: (0, 0)),
            pl.BlockSpec((1, H), lambda i: (0, 0)),
            pl.BlockSpec((1, H), lambda i: (0, 0)),
        ],
        out_specs=pl.BlockSpec((NBLK, R, FB), lambda i: (0, i, 0)),
        out_shape=jax.ShapeDtypeStruct((NBLK, N, FB), jnp.float32),
    )(xpad, m1p, W1l, W1r, b1)


def _tc_phase4(h1b, a2, m1p, W2l, W2r, b2, Wlin, blin):
    R = 2000

    def body(h1_ref, a2_ref, p_ref, wl_ref, wr_ref, b_ref, wo_ref, bo_ref,
             out_ref):
        p = p_ref[0] + p_ref[1]
        inv = 1.0 / jnp.maximum(p[:, 15:16], 1.0)
        wl = wl_ref[...]
        wr = wr_ref[...]
        bb = b_ref[0][None, :]
        acc = jnp.zeros((R, 1), jnp.float32)
        for t in range(T):
            m2 = jnp.concatenate([a2_ref[2 * t], a2_ref[2 * t + 1]], axis=1)
            h1t = jnp.concatenate([h1_ref[2 * t], h1_ref[2 * t + 1]], axis=1)
            h2 = jnp.maximum(
                jnp.dot(m2 * inv, wl, preferred_element_type=jnp.float32)
                + jnp.dot(h1t, wr, preferred_element_type=jnp.float32)
                + bb, 0.0)
            acc = acc + jnp.dot(h2, wo_ref[H * t:H * (t + 1), :],
                                preferred_element_type=jnp.float32)
        out_ref[...] = acc + bo_ref[0, 0]

    return pl.pallas_call(
        body,
        grid=(N // R,),
        in_specs=[
            pl.BlockSpec((NBLK, R, FB), lambda i: (0, i, 0)),
            pl.BlockSpec((NBLK, R, FB), lambda i: (0, i, 0)),
            pl.BlockSpec((2, R, FB), lambda i: (0, i, 0)),
            pl.BlockSpec((H, H), lambda i: (0, 0)),
            pl.BlockSpec((H, H), lambda i: (0, 0)),
            pl.BlockSpec((1, H), lambda i: (0, 0)),
            pl.BlockSpec((H * T, 1), lambda i: (0, 0)),
            pl.BlockSpec((1, 1), lambda i: (0, 0)),
        ],
        out_specs=pl.BlockSpec((R, 1), lambda i: (i, 0)),
        out_shape=jax.ShapeDtypeStruct((N, 1), jnp.float32),
    )(h1b, a2, m1p, W2l, W2r, b2, Wlin, blin)


def kernel(x, edge_index, W1l, b1, W1r, W2l, b2, W2r, Wlin, blin):
    X = x[:, 0, :]
    xpad = jnp.concatenate(
        [X, jnp.zeros((N, FB - T - 1), jnp.float32),
         jnp.ones((N, 1), jnp.float32)], axis=1)
    src_r = edge_index[0].reshape(NCHUNK, CH)
    dst_r = edge_index[1].reshape(NCHUNK, 1, CH)
    zeros_init = jnp.zeros((ROWS_PER_TILE, FB), jnp.float32)

    m1p = _sc_phase1(xpad, src_r, dst_r, zeros_init)
    h1b = _tc_phase2(xpad, m1p, W1l, W1r, b1.reshape(1, H))
    a2 = _sc_phase3(h1b, src_r, dst_r, zeros_init)
    out = _tc_phase4(h1b, a2, m1p, W2l, W2r, b2.reshape(1, H),
                     Wlin, blin.reshape(1, 1))
    return out.reshape(N)


# speedup vs baseline: 20.3965x; 20.3965x over previous
import functools

import jax
import jax.numpy as jnp
from jax import lax
from jax.experimental import pallas as pl
from jax.experimental.pallas import tpu as pltpu
from jax.experimental.pallas import tpu_sc as plsc

N = 100000
E = 1600000
H = 32
T = 10

FB = 16
NBLK = 2 * T
CH = 80
GRP = 5
NG = E // (CH * GRP)
NPAD = 100096
ROWS_PER_TILE = NPAD // 16


def _sc_mesh():
    return plsc.VectorSubcoreMesh(
        core_axis_name="c", subcore_axis_name="s", num_cores=2, num_subcores=16
    )


def _aggregate_pass(table_hbm, src_r, dst_r, acc, zeros_hbm, out_slice,
                    src_buf, dst_buf, rows_buf, sem, sid, g0, ngroups):
    row0 = sid * ROWS_PER_TILE
    pltpu.sync_copy(zeros_hbm, acc.at[pl.ds(row0, ROWS_PER_TILE)])
    plsc.subcore_barrier()

    def group_body(g, _):
        pltpu.sync_copy(src_r.at[g0 + g], src_buf)
        pltpu.sync_copy(dst_r.at[g0 + g], dst_buf)

        def fire(j, _):
            pltpu.async_copy(table_hbm.at[src_buf.at[j]], rows_buf.at[j], sem)
            return 0

        lax.fori_loop(0, GRP, fire, 0)

        def drain(j, _):
            pltpu.make_async_copy(
                table_hbm.at[src_buf.at[j]], rows_buf.at[j], sem
            ).wait()
            pltpu.sync_copy(rows_buf.at[j], acc.at[dst_buf.at[j]], add=True)
            return 0

        lax.fori_loop(0, GRP, drain, 0)
        return 0

    lax.fori_loop(0, ngroups, group_body, 0)
    plsc.subcore_barrier()
    pltpu.sync_copy(acc.at[pl.ds(row0, ROWS_PER_TILE)],
                    out_slice.at[pl.ds(row0, ROWS_PER_TILE)])
    plsc.subcore_barrier()


def _sc_phase1(xpad, src_r, dst_r, zeros_init):
    @functools.partial(
        pl.kernel,
        out_type=jax.ShapeDtypeStruct((2, NPAD, FB), jnp.float32),
        mesh=_sc_mesh(),
        scratch_types=[
            pltpu.VMEM_SHARED((NPAD, FB), jnp.float32),
            pltpu.VMEM((GRP, CH), jnp.int32),
            pltpu.VMEM((GRP, CH), jnp.int32),
            pltpu.VMEM((GRP, CH, FB), jnp.float32),
            pltpu.SemaphoreType.DMA,
        ],
        compiler_params=pltpu.CompilerParams(use_tc_tiling_on_sc=False),
    )
    def k(table_hbm, src_hbm, dst_hbm, zeros_hbm, out_hbm,
          acc, src_buf, dst_buf, rows_buf, sem):
        cid = lax.axis_index("c")
        sid = lax.axis_index("s")
        groups_per_tile = NG // 2 // 16
        g0 = cid * (NG // 2) + sid * groups_per_tile
        _aggregate_pass(table_hbm, src_hbm, dst_hbm, acc, zeros_hbm,
                        out_hbm.at[cid], src_buf, dst_buf, rows_buf, sem,
                        sid, g0, groups_per_tile)

    return k(xpad, src_r, dst_r, zeros_init)


def _sc_phase3(h1b, src_r, dst_r, zeros_init):
    @functools.partial(
        pl.kernel,
        out_type=jax.ShapeDtypeStruct((NBLK, NPAD, FB), jnp.float32),
        mesh=_sc_mesh(),
        scratch_types=[
            pltpu.VMEM_SHARED((NPAD, FB), jnp.float32),
            pltpu.VMEM((GRP, CH), jnp.int32),
            pltpu.VMEM((GRP, CH), jnp.int32),
            pltpu.VMEM((GRP, CH, FB), jnp.float32),
            pltpu.SemaphoreType.DMA,
        ],
        compiler_params=pltpu.CompilerParams(use_tc_tiling_on_sc=False),
    )
    def k(table_hbm, src_hbm, dst_hbm, zeros_hbm, out_hbm,
          acc, src_buf, dst_buf, rows_buf, sem):
        cid = lax.axis_index("c")
        sid = lax.axis_index("s")
        groups_per_tile = NG // 16
        g0 = sid * groups_per_tile
        for j in range(T):
            b = 2 * j
            _aggregate_pass(table_hbm.at[b + cid], src_hbm, dst_hbm, acc,
                            zeros_hbm, out_hbm.at[b + cid], src_buf, dst_buf,
                            rows_buf, sem, sid, g0, groups_per_tile)

    return k(h1b, src_r, dst_r, zeros_init)


def _tc_phase2(xpad, m1p, W1l, W1r, b1):
    R = 2000

    def body(xp_ref, p_ref, wl_ref, wr_ref, b_ref, out_ref):
        p = p_ref[0] + p_ref[1]
        inv = 1.0 / jnp.maximum(p[:, 15:16], 1.0)
        wl = wl_ref[0][None, :]
        wr = wr_ref[0][None, :]
        bb = b_ref[0][None, :]
        for t in range(T):
            m = p[:, t:t + 1] * inv
            xt = xp_ref[:, t:t + 1]
            h = jnp.maximum(m * wl + xt * wr + bb, 0.0)
            out_ref[2 * t] = h[:, :FB]
            out_ref[2 * t + 1] = h[:, FB:]

    return pl.pallas_call(
        body,
        grid=(N // R,),
        in_specs=[
            pl.BlockSpec((R, FB), lambda i: (i, 0)),
            pl.BlockSpec((2, R, FB), lambda i: (0, i, 0)),
            pl.BlockSpec((1, H), lambda i: (0, 0)),
            pl.BlockSpec((1, H), lambda i: (0, 0)),
            pl.BlockSpec((1, H), lambda i: (0, 0)),
        ],
        out_specs=pl.BlockSpec((NBLK, R, FB), lambda i: (0, i, 0)),
        out_shape=jax.ShapeDtypeStruct((NBLK, NPAD, FB), jnp.float32),
    )(xpad, m1p, W1l, W1r, b1)


def _tc_phase4(h1b, a2, m1p, W2l, W2r, b2, Wlin, blin):
    R = 1000

    def body(h1_ref, a2_ref, p_ref, wl_ref, wr_ref, b_ref, wo_ref, bo_ref,
             out_ref):
        p = p_ref[0] + p_ref[1]
        inv = 1.0 / jnp.maximum(p[:, 15:16], 1.0)
        wl = wl_ref[...]
        wr = wr_ref[...]
        bb = b_ref[0][None, :]
        acc = jnp.zeros((R, 1), jnp.float32)
        for t in range(T):
            m2 = jnp.concatenate([a2_ref[2 * t], a2_ref[2 * t + 1]], axis=1)
            h1t = jnp.concatenate([h1_ref[2 * t], h1_ref[2 * t + 1]], axis=1)
            h2 = jnp.maximum(
                jnp.dot(m2 * inv, wl, preferred_element_type=jnp.float32)
                + jnp.dot(h1t, wr, preferred_element_type=jnp.float32)
                + bb, 0.0)
            acc = acc + jnp.dot(h2, wo_ref[H * t:H * (t + 1), :],
                                preferred_element_type=jnp.float32)
        out_ref[...] = acc + bo_ref[0, 0]

    return pl.pallas_call(
        body,
        grid=(N // R,),
        in_specs=[
            pl.BlockSpec((NBLK, R, FB), lambda i: (0, i, 0)),
            pl.BlockSpec((NBLK, R, FB), lambda i: (0, i, 0)),
            pl.BlockSpec((2, R, FB), lambda i: (0, i, 0)),
            pl.BlockSpec((H, H), lambda i: (0, 0)),
            pl.BlockSpec((H, H), lambda i: (0, 0)),
            pl.BlockSpec((1, H), lambda i: (0, 0)),
            pl.BlockSpec((H * T, 1), lambda i: (0, 0)),
            pl.BlockSpec((1, 1), lambda i: (0, 0)),
        ],
        out_specs=pl.BlockSpec((R, 1), lambda i: (i, 0)),
        out_shape=jax.ShapeDtypeStruct((N, 1), jnp.float32),
    )(h1b, a2, m1p, W2l, W2r, b2, Wlin, blin)


def kernel(x, edge_index, W1l, b1, W1r, W2l, b2, W2r, Wlin, blin):
    X = x[:, 0, :]
    xpad = jnp.concatenate(
        [X, jnp.zeros((N, FB - T - 1), jnp.float32),
         jnp.ones((N, 1), jnp.float32)], axis=1)
    xpad = jnp.concatenate(
        [xpad, jnp.zeros((NPAD - N, FB), jnp.float32)], axis=0)
    src_r = edge_index[0].reshape(NG, GRP, CH)
    dst_r = edge_index[1].reshape(NG, GRP, CH)
    zeros_init = jnp.zeros((ROWS_PER_TILE, FB), jnp.float32)

    m1p = _sc_phase1(xpad, src_r, dst_r, zeros_init)
    h1b = _tc_phase2(xpad, m1p, W1l, W1r, b1.reshape(1, H))
    a2 = _sc_phase3(h1b, src_r, dst_r, zeros_init)
    out = _tc_phase4(h1b, a2, m1p, W2l, W2r, b2.reshape(1, H),
                     Wlin, blin.reshape(1, 1))
    return out.reshape(N)


# speedup vs baseline: 24.7336x; 1.2126x over previous
import functools

import jax
import jax.numpy as jnp
from jax import lax
from jax.experimental import pallas as pl
from jax.experimental.pallas import tpu as pltpu
from jax.experimental.pallas import tpu_sc as plsc

N = 100000
E = 1600000
H = 32
T = 10

FB = 16
NBLK = 2 * T
CH = 128
GRP = 5
EPAD = 1638400
NG = EPAD // (CH * GRP)
NPAD = 100096
ROWS_PER_TILE = NPAD // 16


def _sc_mesh():
    return plsc.VectorSubcoreMesh(
        core_axis_name="c", subcore_axis_name="s", num_cores=2, num_subcores=16
    )


def _aggregate_pass(table_hbm, src_r, dst_r, acc, zeros_hbm, out_slice,
                    src_buf, dst_buf, rows_buf, sem0, sem1, sid, g0, ngroups):
    row0 = sid * ROWS_PER_TILE
    pltpu.sync_copy(zeros_hbm, acc.at[pl.ds(row0, ROWS_PER_TILE)])
    plsc.subcore_barrier()

    sems = (sem0, sem1)

    def load_fire(g, p):
        pltpu.sync_copy(src_r.at[g0 + g], src_buf.at[p])
        pltpu.sync_copy(dst_r.at[g0 + g], dst_buf.at[p])

        def fire(j, _):
            pltpu.async_copy(table_hbm.at[src_buf.at[p].at[j]],
                             rows_buf.at[p].at[j], sems[p])
            return 0

        lax.fori_loop(0, GRP, fire, 0)

    def drain(p):
        def d(j, _):
            pltpu.make_async_copy(
                table_hbm.at[src_buf.at[p].at[j]], rows_buf.at[p].at[j],
                sems[p]
            ).wait()
            pltpu.sync_copy(rows_buf.at[p].at[j],
                            acc.at[dst_buf.at[p].at[j]], add=True)
            return 0

        lax.fori_loop(0, GRP, d, 0)

    load_fire(0, 0)

    def body(h, _):
        load_fire(2 * h + 1, 1)
        drain(0)
        load_fire(2 * h + 2, 0)
        drain(1)
        return 0

    lax.fori_loop(0, ngroups // 2 - 1, body, 0)
    load_fire(ngroups - 1, 1)
    drain(0)
    drain(1)
    plsc.subcore_barrier()
    pltpu.sync_copy(acc.at[pl.ds(row0, ROWS_PER_TILE)],
                    out_slice.at[pl.ds(row0, ROWS_PER_TILE)])
    plsc.subcore_barrier()


def _sc_phase1(xpad, src_r, dst_r, zeros_init):
    @functools.partial(
        pl.kernel,
        out_type=jax.ShapeDtypeStruct((2, NPAD, FB), jnp.float32),
        mesh=_sc_mesh(),
        scratch_types=[
            pltpu.VMEM_SHARED((NPAD, FB), jnp.float32),
            pltpu.VMEM((2, GRP, CH), jnp.int32),
            pltpu.VMEM((2, GRP, CH), jnp.int32),
            pltpu.VMEM((2, GRP, CH, FB), jnp.float32),
            pltpu.SemaphoreType.DMA,
            pltpu.SemaphoreType.DMA,
        ],
        compiler_params=pltpu.CompilerParams(use_tc_tiling_on_sc=False),
    )
    def k(table_hbm, src_hbm, dst_hbm, zeros_hbm, out_hbm,
          acc, src_buf, dst_buf, rows_buf, sem0, sem1):
        cid = lax.axis_index("c")
        sid = lax.axis_index("s")
        groups_per_tile = NG // 2 // 16
        g0 = cid * (NG // 2) + sid * groups_per_tile
        _aggregate_pass(table_hbm, src_hbm, dst_hbm, acc, zeros_hbm,
                        out_hbm.at[cid], src_buf, dst_buf, rows_buf,
                        sem0, sem1, sid, g0, groups_per_tile)

    return k(xpad, src_r, dst_r, zeros_init)


def _sc_phase3(h1b, src_r, dst_r, zeros_init):
    @functools.partial(
        pl.kernel,
        out_type=jax.ShapeDtypeStruct((NBLK, NPAD, FB), jnp.float32),
        mesh=_sc_mesh(),
        scratch_types=[
            pltpu.VMEM_SHARED((NPAD, FB), jnp.float32),
            pltpu.VMEM((2, GRP, CH), jnp.int32),
            pltpu.VMEM((2, GRP, CH), jnp.int32),
            pltpu.VMEM((2, GRP, CH, FB), jnp.float32),
            pltpu.SemaphoreType.DMA,
            pltpu.SemaphoreType.DMA,
        ],
        compiler_params=pltpu.CompilerParams(use_tc_tiling_on_sc=False),
    )
    def k(table_hbm, src_hbm, dst_hbm, zeros_hbm, out_hbm,
          acc, src_buf, dst_buf, rows_buf, sem0, sem1):
        cid = lax.axis_index("c")
        sid = lax.axis_index("s")
        groups_per_tile = NG // 16
        g0 = sid * groups_per_tile
        for j in range(T):
            b = 2 * j
            _aggregate_pass(table_hbm.at[b + cid], src_hbm, dst_hbm, acc,
                            zeros_hbm, out_hbm.at[b + cid], src_buf, dst_buf,
                            rows_buf, sem0, sem1, sid, g0, groups_per_tile)

    return k(h1b, src_r, dst_r, zeros_init)


def _tc_phase2(xpad, m1p, W1l, W1r, b1):
    R = 2000

    def body(xp_ref, p_ref, wl_ref, wr_ref, b_ref, out_ref):
        p = p_ref[0] + p_ref[1]
        inv = 1.0 / jnp.maximum(p[:, 15:16], 1.0)
        wl = wl_ref[0][None, :]
        wr = wr_ref[0][None, :]
        bb = b_ref[0][None, :]
        for t in range(T):
            m = p[:, t:t + 1] * inv
            xt = xp_ref[:, t:t + 1]
            h = jnp.maximum(m * wl + xt * wr + bb, 0.0)
            out_ref[2 * t] = h[:, :FB]
            out_ref[2 * t + 1] = h[:, FB:]

    return pl.pallas_call(
        body,
        grid=(N // R,),
        in_specs=[
            pl.BlockSpec((R, FB), lambda i: (i, 0)),
            pl.BlockSpec((2, R, FB), lambda i: (0, i, 0)),
            pl.BlockSpec((1, H), lambda i: (0, 0)),
            pl.BlockSpec((1, H), lambda i: (0, 0)),
            pl.BlockSpec((1, H), lambda i: (0, 0)),
        ],
        out_specs=pl.BlockSpec((NBLK, R, FB), lambda i: (0, i, 0)),
        out_shape=jax.ShapeDtypeStruct((NBLK, NPAD, FB), jnp.float32),
    )(xpad, m1p, W1l, W1r, b1)


def _tc_phase4(h1b, a2, m1p, W2l, W2r, b2, Wlin, blin):
    R = 1000

    def body(h1_ref, a2_ref, p_ref, wl_ref, wr_ref, b_ref, wo_ref, bo_ref,
             out_ref):
        p = p_ref[0] + p_ref[1]
        inv = 1.0 / jnp.maximum(p[:, 15:16], 1.0)
        wl = wl_ref[...]
        wr = wr_ref[...]
        bb = b_ref[0][None, :]
        acc = jnp.zeros((R, 1), jnp.float32)
        for t in range(T):
            m2 = jnp.concatenate([a2_ref[2 * t], a2_ref[2 * t + 1]], axis=1)
            h1t = jnp.concatenate([h1_ref[2 * t], h1_ref[2 * t + 1]], axis=1)
            h2 = jnp.maximum(
                jnp.dot(m2 * inv, wl, preferred_element_type=jnp.float32)
                + jnp.dot(h1t, wr, preferred_element_type=jnp.float32)
                + bb, 0.0)
            acc = acc + jnp.dot(h2, wo_ref[H * t:H * (t + 1), :],
                                preferred_element_type=jnp.float32)
        out_ref[...] = acc + bo_ref[0, 0]

    return pl.pallas_call(
        body,
        grid=(N // R,),
        in_specs=[
            pl.BlockSpec((NBLK, R, FB), lambda i: (0, i, 0)),
            pl.BlockSpec((NBLK, R, FB), lambda i: (0, i, 0)),
            pl.BlockSpec((2, R, FB), lambda i: (0, i, 0)),
            pl.BlockSpec((H, H), lambda i: (0, 0)),
            pl.BlockSpec((H, H), lambda i: (0, 0)),
            pl.BlockSpec((1, H), lambda i: (0, 0)),
            pl.BlockSpec((H * T, 1), lambda i: (0, 0)),
            pl.BlockSpec((1, 1), lambda i: (0, 0)),
        ],
        out_specs=pl.BlockSpec((R, 1), lambda i: (i, 0)),
        out_shape=jax.ShapeDtypeStruct((N, 1), jnp.float32),
    )(h1b, a2, m1p, W2l, W2r, b2, Wlin, blin)


def kernel(x, edge_index, W1l, b1, W1r, W2l, b2, W2r, Wlin, blin):
    X = x[:, 0, :]
    xpad = jnp.concatenate(
        [X, jnp.zeros((N, FB - T - 1), jnp.float32),
         jnp.ones((N, 1), jnp.float32)], axis=1)
    xpad = jnp.concatenate(
        [xpad, jnp.zeros((NPAD - N, FB), jnp.float32)], axis=0)
    epad = jnp.full((EPAD - E,), NPAD - 1, jnp.int32)
    src_r = jnp.concatenate([edge_index[0], epad]).reshape(NG, GRP, CH)
    dst_r = jnp.concatenate([edge_index[1], epad]).reshape(NG, GRP, CH)
    zeros_init = jnp.zeros((ROWS_PER_TILE, FB), jnp.float32)

    m1p = _sc_phase1(xpad, src_r, dst_r, zeros_init)
    h1b = _tc_phase2(xpad, m1p, W1l, W1r, b1.reshape(1, H))
    a2 = _sc_phase3(h1b, src_r, dst_r, zeros_init)
    out = _tc_phase4(h1b, a2, m1p, W2l, W2r, b2.reshape(1, H),
                     Wlin, blin.reshape(1, 1))
    return out.reshape(N)


# speedup vs baseline: 36.0968x; 1.4594x over previous
import functools

import jax
import jax.numpy as jnp
from jax import lax
from jax.experimental import pallas as pl
from jax.experimental.pallas import tpu as pltpu
from jax.experimental.pallas import tpu_sc as plsc

N = 100000
E = 1600000
H = 32
T = 10

FB = 16
NBLK = 2 * T
CH = 128
GRP = 8
EPAD = 1638400
NG = EPAD // (CH * GRP)
NPAD = 100096
ROWS_PER_TILE = NPAD // 16


def _sc_mesh():
    return plsc.VectorSubcoreMesh(
        core_axis_name="c", subcore_axis_name="s", num_cores=2, num_subcores=16
    )


def _aggregate_pass(table_hbm, src_r, dst_r, acc, zeros_hbm, out_slice,
                    src_buf, dst_buf, rows_buf, sems, sid, g0, ngroups):
    row0 = sid * ROWS_PER_TILE
    pltpu.sync_copy(zeros_hbm, acc.at[pl.ds(row0, ROWS_PER_TILE)])
    plsc.subcore_barrier()

    def load_idx(g, p):
        pltpu.sync_copy(src_r.at[g0 + g], src_buf.at[p])
        pltpu.sync_copy(dst_r.at[g0 + g], dst_buf.at[p])

    def fire(p, j):
        pltpu.async_copy(table_hbm.at[src_buf.at[p].at[j]],
                         rows_buf.at[j], sems.at[j])

    def wait_scatter(p, j):
        pltpu.make_async_copy(table_hbm.at[src_buf.at[p].at[j]],
                              rows_buf.at[j], sems.at[j]).wait()
        pltpu.sync_copy(rows_buf.at[j], acc.at[dst_buf.at[p].at[j]],
                        add=True)

    load_idx(0, 0)
    for j in range(GRP):
        fire(0, j)

    def body(h, _):
        load_idx(2 * h + 1, 1)
        for j in range(GRP):
            wait_scatter(0, j)
            fire(1, j)
        load_idx(2 * h + 2, 0)
        for j in range(GRP):
            wait_scatter(1, j)
            fire(0, j)
        return 0

    lax.fori_loop(0, (ngroups - 2) // 2, body, 0)
    load_idx(ngroups - 1, 1)
    for j in range(GRP):
        wait_scatter(0, j)
        fire(1, j)
    for j in range(GRP):
        wait_scatter(1, j)
    plsc.subcore_barrier()
    pltpu.sync_copy(acc.at[pl.ds(row0, ROWS_PER_TILE)],
                    out_slice.at[pl.ds(row0, ROWS_PER_TILE)])
    plsc.subcore_barrier()


def _sc_phase1(xpad, src_r, dst_r, zeros_init):
    @functools.partial(
        pl.kernel,
        out_type=jax.ShapeDtypeStruct((2, NPAD, FB), jnp.float32),
        mesh=_sc_mesh(),
        scratch_types=[
            pltpu.VMEM_SHARED((NPAD, FB), jnp.float32),
            pltpu.VMEM((2, GRP, CH), jnp.int32),
            pltpu.VMEM((2, GRP, CH), jnp.int32),
            pltpu.VMEM((GRP, CH, FB), jnp.float32),
            pltpu.SemaphoreType.DMA((GRP,)),
        ],
        compiler_params=pltpu.CompilerParams(use_tc_tiling_on_sc=False),
    )
    def k(table_hbm, src_hbm, dst_hbm, zeros_hbm, out_hbm,
          acc, src_buf, dst_buf, rows_buf, sems):
        cid = lax.axis_index("c")
        sid = lax.axis_index("s")
        groups_per_tile = NG // 2 // 16
        g0 = cid * (NG // 2) + sid * groups_per_tile
        _aggregate_pass(table_hbm, src_hbm, dst_hbm, acc, zeros_hbm,
                        out_hbm.at[cid], src_buf, dst_buf, rows_buf,
                        sems, sid, g0, groups_per_tile)

    return k(xpad, src_r, dst_r, zeros_init)


def _sc_phase3(h1b, src_r, dst_r, zeros_init):
    @functools.partial(
        pl.kernel,
        out_type=jax.ShapeDtypeStruct((NBLK, NPAD, FB), jnp.float32),
        mesh=_sc_mesh(),
        scratch_types=[
            pltpu.VMEM_SHARED((NPAD, FB), jnp.float32),
            pltpu.VMEM((2, GRP, CH), jnp.int32),
            pltpu.VMEM((2, GRP, CH), jnp.int32),
            pltpu.VMEM((GRP, CH, FB), jnp.float32),
            pltpu.SemaphoreType.DMA((GRP,)),
        ],
        compiler_params=pltpu.CompilerParams(use_tc_tiling_on_sc=False),
    )
    def k(table_hbm, src_hbm, dst_hbm, zeros_hbm, out_hbm,
          acc, src_buf, dst_buf, rows_buf, sems):
        cid = lax.axis_index("c")
        sid = lax.axis_index("s")
        groups_per_tile = NG // 16
        g0 = sid * groups_per_tile
        for j in range(T):
            b = 2 * j
            _aggregate_pass(table_hbm.at[b + cid], src_hbm, dst_hbm, acc,
                            zeros_hbm, out_hbm.at[b + cid], src_buf, dst_buf,
                            rows_buf, sems, sid, g0, groups_per_tile)

    return k(h1b, src_r, dst_r, zeros_init)


def _tc_phase2(xpad, m1p, W1l, W1r, b1):
    R = 2000

    def body(xp_ref, p_ref, wl_ref, wr_ref, b_ref, out_ref):
        p = p_ref[0] + p_ref[1]
        inv = 1.0 / jnp.maximum(p[:, 15:16], 1.0)
        wl = wl_ref[0][None, :]
        wr = wr_ref[0][None, :]
        bb = b_ref[0][None, :]
        for t in range(T):
            m = p[:, t:t + 1] * inv
            xt = xp_ref[:, t:t + 1]
            h = jnp.maximum(m * wl + xt * wr + bb, 0.0)
            out_ref[2 * t] = h[:, :FB]
            out_ref[2 * t + 1] = h[:, FB:]

    return pl.pallas_call(
        body,
        grid=(N // R,),
        in_specs=[
            pl.BlockSpec((R, FB), lambda i: (i, 0)),
            pl.BlockSpec((2, R, FB), lambda i: (0, i, 0)),
            pl.BlockSpec((1, H), lambda i: (0, 0)),
            pl.BlockSpec((1, H), lambda i: (0, 0)),
            pl.BlockSpec((1, H), lambda i: (0, 0)),
        ],
        out_specs=pl.BlockSpec((NBLK, R, FB), lambda i: (0, i, 0)),
        out_shape=jax.ShapeDtypeStruct((NBLK, NPAD, FB), jnp.float32),
    )(xpad, m1p, W1l, W1r, b1)


def _tc_phase4(h1b, a2, m1p, W2l, W2r, b2, Wlin, blin):
    R = 1000

    def body(h1_ref, a2_ref, p_ref, wl_ref, wr_ref, b_ref, wo_ref, bo_ref,
             out_ref):
        p = p_ref[0] + p_ref[1]
        inv = 1.0 / jnp.maximum(p[:, 15:16], 1.0)
        wl = wl_ref[...]
        wr = wr_ref[...]
        bb = b_ref[0][None, :]
        acc = jnp.zeros((R, 1), jnp.float32)
        for t in range(T):
            m2 = jnp.concatenate([a2_ref[2 * t], a2_ref[2 * t + 1]], axis=1)
            h1t = jnp.concatenate([h1_ref[2 * t], h1_ref[2 * t + 1]], axis=1)
            h2 = jnp.maximum(
                jnp.dot(m2 * inv, wl, preferred_element_type=jnp.float32)
                + jnp.dot(h1t, wr, preferred_element_type=jnp.float32)
                + bb, 0.0)
            acc = acc + jnp.dot(h2, wo_ref[H * t:H * (t + 1), :],
                                preferred_element_type=jnp.float32)
        out_ref[...] = acc + bo_ref[0, 0]

    return pl.pallas_call(
        body,
        grid=(N // R,),
        in_specs=[
            pl.BlockSpec((NBLK, R, FB), lambda i: (0, i, 0)),
            pl.BlockSpec((NBLK, R, FB), lambda i: (0, i, 0)),
            pl.BlockSpec((2, R, FB), lambda i: (0, i, 0)),
            pl.BlockSpec((H, H), lambda i: (0, 0)),
            pl.BlockSpec((H, H), lambda i: (0, 0)),
            pl.BlockSpec((1, H), lambda i: (0, 0)),
            pl.BlockSpec((H * T, 1), lambda i: (0, 0)),
            pl.BlockSpec((1, 1), lambda i: (0, 0)),
        ],
        out_specs=pl.BlockSpec((R, 1), lambda i: (i, 0)),
        out_shape=jax.ShapeDtypeStruct((N, 1), jnp.float32),
    )(h1b, a2, m1p, W2l, W2r, b2, Wlin, blin)


def kernel(x, edge_index, W1l, b1, W1r, W2l, b2, W2r, Wlin, blin):
    X = x[:, 0, :]
    xpad = jnp.concatenate(
        [X, jnp.zeros((N, FB - T - 1), jnp.float32),
         jnp.ones((N, 1), jnp.float32)], axis=1)
    xpad = jnp.concatenate(
        [xpad, jnp.zeros((NPAD - N, FB), jnp.float32)], axis=0)
    epad = N + jnp.arange(EPAD - E, dtype=jnp.int32) % (NPAD - N)
    src_r = jnp.concatenate([edge_index[0], epad]).reshape(NG, GRP, CH)
    dst_r = jnp.concatenate([edge_index[1], epad]).reshape(NG, GRP, CH)
    zeros_init = jnp.zeros((ROWS_PER_TILE, FB), jnp.float32)

    m1p = _sc_phase1(xpad, src_r, dst_r, zeros_init)
    h1b = _tc_phase2(xpad, m1p, W1l, W1r, b1.reshape(1, H))
    a2 = _sc_phase3(h1b, src_r, dst_r, zeros_init)
    out = _tc_phase4(h1b, a2, m1p, W2l, W2r, b2.reshape(1, H),
                     Wlin, blin.reshape(1, 1))
    return out.reshape(N)


# speedup vs baseline: 37.6362x; 1.0426x over previous
import functools

import jax
import jax.numpy as jnp
from jax import lax
from jax.experimental import pallas as pl
from jax.experimental.pallas import tpu as pltpu
from jax.experimental.pallas import tpu_sc as plsc

N = 100000
E = 1600000
H = 32
T = 10

FB = 16
NBLK = 2 * T
CH = 128
GRP = 10
EPAD = 1638400
NG = EPAD // (CH * GRP)
NPAD = 100096
ROWS_PER_TILE = NPAD // 16


def _sc_mesh():
    return plsc.VectorSubcoreMesh(
        core_axis_name="c", subcore_axis_name="s", num_cores=2, num_subcores=16
    )


def _aggregate_pass(table_hbm, src_r, dst_r, acc, zeros_hbm, out_slice,
                    src_buf, dst_buf, rows_buf, sems, sid, g0, ngroups):
    row0 = sid * ROWS_PER_TILE
    pltpu.sync_copy(zeros_hbm, acc.at[pl.ds(row0, ROWS_PER_TILE)])
    plsc.subcore_barrier()

    def load_idx(g, p):
        pltpu.sync_copy(src_r.at[g0 + g], src_buf.at[p])
        pltpu.sync_copy(dst_r.at[g0 + g], dst_buf.at[p])

    def fire(p, j):
        pltpu.async_copy(table_hbm.at[src_buf.at[p].at[j]],
                         rows_buf.at[j], sems.at[j])

    def wait_scatter(p, j):
        pltpu.make_async_copy(table_hbm.at[src_buf.at[p].at[j]],
                              rows_buf.at[j], sems.at[j]).wait()
        pltpu.sync_copy(rows_buf.at[j], acc.at[dst_buf.at[p].at[j]],
                        add=True)

    load_idx(0, 0)
    for j in range(GRP):
        fire(0, j)

    def body(h, _):
        load_idx(2 * h + 1, 1)
        for j in range(GRP):
            wait_scatter(0, j)
            fire(1, j)
        load_idx(2 * h + 2, 0)
        for j in range(GRP):
            wait_scatter(1, j)
            fire(0, j)
        return 0

    lax.fori_loop(0, (ngroups - 2) // 2, body, 0)
    load_idx(ngroups - 1, 1)
    for j in range(GRP):
        wait_scatter(0, j)
        fire(1, j)
    for j in range(GRP):
        wait_scatter(1, j)
    plsc.subcore_barrier()
    pltpu.sync_copy(acc.at[pl.ds(row0, ROWS_PER_TILE)],
                    out_slice.at[pl.ds(row0, ROWS_PER_TILE)])
    plsc.subcore_barrier()


def _sc_phase1(xpad, src_r, dst_r, zeros_init):
    @functools.partial(
        pl.kernel,
        out_type=jax.ShapeDtypeStruct((2, NPAD, FB), jnp.float32),
        mesh=_sc_mesh(),
        scratch_types=[
            pltpu.VMEM_SHARED((NPAD, FB), jnp.float32),
            pltpu.VMEM((2, GRP, CH), jnp.int32),
            pltpu.VMEM((2, GRP, CH), jnp.int32),
            pltpu.VMEM((GRP, CH, FB), jnp.float32),
            pltpu.SemaphoreType.DMA((GRP,)),
        ],
        compiler_params=pltpu.CompilerParams(use_tc_tiling_on_sc=False),
    )
    def k(table_hbm, src_hbm, dst_hbm, zeros_hbm, out_hbm,
          acc, src_buf, dst_buf, rows_buf, sems):
        cid = lax.axis_index("c")
        sid = lax.axis_index("s")
        groups_per_tile = NG // 2 // 16
        g0 = cid * (NG // 2) + sid * groups_per_tile
        _aggregate_pass(table_hbm, src_hbm, dst_hbm, acc, zeros_hbm,
                        out_hbm.at[cid], src_buf, dst_buf, rows_buf,
                        sems, sid, g0, groups_per_tile)

    return k(xpad, src_r, dst_r, zeros_init)


def _sc_phase3(h1b, src_r, dst_r, zeros_init):
    @functools.partial(
        pl.kernel,
        out_type=jax.ShapeDtypeStruct((NBLK, NPAD, FB), jnp.float32),
        mesh=_sc_mesh(),
        scratch_types=[
            pltpu.VMEM_SHARED((NPAD, FB), jnp.float32),
            pltpu.VMEM((2, GRP, CH), jnp.int32),
            pltpu.VMEM((2, GRP, CH), jnp.int32),
            pltpu.VMEM((GRP, CH, FB), jnp.float32),
            pltpu.SemaphoreType.DMA((GRP,)),
        ],
        compiler_params=pltpu.CompilerParams(use_tc_tiling_on_sc=False),
    )
    def k(table_hbm, src_hbm, dst_hbm, zeros_hbm, out_hbm,
          acc, src_buf, dst_buf, rows_buf, sems):
        cid = lax.axis_index("c")
        sid = lax.axis_index("s")
        groups_per_tile = NG // 16
        g0 = sid * groups_per_tile
        for j in range(T):
            b = 2 * j
            _aggregate_pass(table_hbm.at[b + cid], src_hbm, dst_hbm, acc,
                            zeros_hbm, out_hbm.at[b + cid], src_buf, dst_buf,
                            rows_buf, sems, sid, g0, groups_per_tile)

    return k(h1b, src_r, dst_r, zeros_init)


def _tc_phase2(xpad, m1p, W1l, W1r, b1):
    R = 2000

    def body(xp_ref, p_ref, wl_ref, wr_ref, b_ref, out_ref):
        p = p_ref[0] + p_ref[1]
        inv = 1.0 / jnp.maximum(p[:, 15:16], 1.0)
        wl = wl_ref[0][None, :]
        wr = wr_ref[0][None, :]
        bb = b_ref[0][None, :]
        for t in range(T):
            m = p[:, t:t + 1] * inv
            xt = xp_ref[:, t:t + 1]
            h = jnp.maximum(m * wl + xt * wr + bb, 0.0)
            out_ref[2 * t] = h[:, :FB]
            out_ref[2 * t + 1] = h[:, FB:]

    return pl.pallas_call(
        body,
        grid=(N // R,),
        in_specs=[
            pl.BlockSpec((R, FB), lambda i: (i, 0)),
            pl.BlockSpec((2, R, FB), lambda i: (0, i, 0)),
            pl.BlockSpec((1, H), lambda i: (0, 0)),
            pl.BlockSpec((1, H), lambda i: (0, 0)),
            pl.BlockSpec((1, H), lambda i: (0, 0)),
        ],
        out_specs=pl.BlockSpec((NBLK, R, FB), lambda i: (0, i, 0)),
        out_shape=jax.ShapeDtypeStruct((NBLK, NPAD, FB), jnp.float32),
    )(xpad, m1p, W1l, W1r, b1)


def _tc_phase4(h1b, a2, m1p, W2l, W2r, b2, Wlin, blin):
    R = 1000

    def body(h1_ref, a2_ref, p_ref, wl_ref, wr_ref, b_ref, wo_ref, bo_ref,
             out_ref):
        p = p_ref[0] + p_ref[1]
        inv = 1.0 / jnp.maximum(p[:, 15:16], 1.0)
        wl = wl_ref[...]
        wr = wr_ref[...]
        bb = b_ref[0][None, :]
        acc = jnp.zeros((R, 1), jnp.float32)
        for t in range(T):
            m2 = jnp.concatenate([a2_ref[2 * t], a2_ref[2 * t + 1]], axis=1)
            h1t = jnp.concatenate([h1_ref[2 * t], h1_ref[2 * t + 1]], axis=1)
            h2 = jnp.maximum(
                jnp.dot(m2 * inv, wl, preferred_element_type=jnp.float32)
                + jnp.dot(h1t, wr, preferred_element_type=jnp.float32)
                + bb, 0.0)
            acc = acc + jnp.dot(h2, wo_ref[H * t:H * (t + 1), :],
                                preferred_element_type=jnp.float32)
        out_ref[...] = acc + bo_ref[0, 0]

    return pl.pallas_call(
        body,
        grid=(N // R,),
        in_specs=[
            pl.BlockSpec((NBLK, R, FB), lambda i: (0, i, 0)),
            pl.BlockSpec((NBLK, R, FB), lambda i: (0, i, 0)),
            pl.BlockSpec((2, R, FB), lambda i: (0, i, 0)),
            pl.BlockSpec((H, H), lambda i: (0, 0)),
            pl.BlockSpec((H, H), lambda i: (0, 0)),
            pl.BlockSpec((1, H), lambda i: (0, 0)),
            pl.BlockSpec((H * T, 1), lambda i: (0, 0)),
            pl.BlockSpec((1, 1), lambda i: (0, 0)),
        ],
        out_specs=pl.BlockSpec((R, 1), lambda i: (i, 0)),
        out_shape=jax.ShapeDtypeStruct((N, 1), jnp.float32),
    )(h1b, a2, m1p, W2l, W2r, b2, Wlin, blin)


def kernel(x, edge_index, W1l, b1, W1r, W2l, b2, W2r, Wlin, blin):
    X = x[:, 0, :]
    xpad = jnp.concatenate(
        [X, jnp.zeros((N, FB - T - 1), jnp.float32),
         jnp.ones((N, 1), jnp.float32)], axis=1)
    xpad = jnp.concatenate(
        [xpad, jnp.zeros((NPAD - N, FB), jnp.float32)], axis=0)
    epad = N + jnp.arange(EPAD - E, dtype=jnp.int32) % (NPAD - N)
    src_r = jnp.concatenate([edge_index[0], epad]).reshape(NG, GRP, CH)
    dst_r = jnp.concatenate([edge_index[1], epad]).reshape(NG, GRP, CH)
    zeros_init = jnp.zeros((ROWS_PER_TILE, FB), jnp.float32)

    m1p = _sc_phase1(xpad, src_r, dst_r, zeros_init)
    h1b = _tc_phase2(xpad, m1p, W1l, W1r, b1.reshape(1, H))
    a2 = _sc_phase3(h1b, src_r, dst_r, zeros_init)
    out = _tc_phase4(h1b, a2, m1p, W2l, W2r, b2.reshape(1, H),
                     Wlin, blin.reshape(1, 1))
    return out.reshape(N)
